# baseline (speedup 1.0000x reference)
import functools

import jax
import jax.numpy as jnp
from jax import lax
from jax.experimental import pallas as pl
from jax.experimental.pallas import tpu as pltpu
from jax.experimental.pallas import tpu_sc as plsc

D_MODEL = 1024
L_SEQ = 4096
NUM_EMB = 5
TEXT_ID = 0
VISUAL_ID = 3

L_SC = 2048
L_TC = L_SEQ - L_SC

NUM_CORES = 2
NUM_SUBCORES = 16
LANES = 16
NUM_WORKERS = NUM_CORES * NUM_SUBCORES
ROWS_PER_WORKER = L_SC // NUM_WORKERS
CHUNK = 32
NCHUNK = ROWS_PER_WORKER // CHUNK
SLICES = D_MODEL // LANES

_MESH = plsc.VectorSubcoreMesh(core_axis_name="c", subcore_axis_name="s")


@functools.partial(
    pl.kernel,
    out_type=jax.ShapeDtypeStruct((L_SC, 1, D_MODEL), jnp.float32),
    mesh=_MESH,
    scratch_types=[
        pltpu.VMEM((NUM_EMB, D_MODEL), jnp.float32),
        pltpu.VMEM((LANES,), jnp.int32),
        pltpu.VMEM((CHUNK, 1, D_MODEL), jnp.float32),
        pltpu.VMEM((CHUNK, 1, D_MODEL), jnp.float32),
        pltpu.SemaphoreType.DMA,
        pltpu.SemaphoreType.DMA,
    ],
)
def _emb_lookup_sc(table_hbm, ntxt_hbm, out_hbm, tab_v, ntxt_v, buf0, buf1, sem0, sem1):
    wid = lax.axis_index("s") * NUM_CORES + lax.axis_index("c")
    base = wid * ROWS_PER_WORKER
    pltpu.sync_copy(ntxt_hbm, ntxt_v)
    pltpu.sync_copy(table_hbm, tab_v)
    ntxt = ntxt_v[...]

    def fill(buf, cbase):
        conds = [jnp.full((LANES,), cbase + r, jnp.int32) < ntxt for r in range(CHUNK)]

        def body(s, carry):
            off = s * LANES
            e0 = tab_v[TEXT_ID, pl.ds(off, LANES)]
            e3 = tab_v[VISUAL_ID, pl.ds(off, LANES)]
            for r in range(CHUNK):
                buf[r, 0, pl.ds(off, LANES)] = jnp.where(conds[r], e0, e3)
            return carry

        lax.fori_loop(0, SLICES, body, 0)

    bufs = (buf0, buf1)
    sems = (sem0, sem1)
    handles = [None, None]
    for c in range(NCHUNK):
        b = c % 2
        if handles[b] is not None:
            handles[b].wait()
        cbase = base + c * CHUNK
        fill(bufs[b], cbase)
        handles[b] = pltpu.async_copy(bufs[b], out_hbm.at[pl.ds(cbase, CHUNK)], sems[b])
    for h in handles:
        if h is not None:
            h.wait()


TC_BLOCK = 256


def _emb_lookup_tc_body(ntxt_ref, emb_ref, out_ref):
    i = pl.program_id(0)
    rows = lax.broadcasted_iota(jnp.int32, (TC_BLOCK, 1, 1), 0) + (L_SC + i * TC_BLOCK)
    mask = rows < ntxt_ref[0]
    e0 = emb_ref[TEXT_ID, :].reshape(1, 1, D_MODEL)
    e3 = emb_ref[VISUAL_ID, :].reshape(1, 1, D_MODEL)
    out_ref[...] = jnp.where(mask, e0, e3)


_emb_lookup_tc = pl.pallas_call(
    _emb_lookup_tc_body,
    grid_spec=pltpu.PrefetchScalarGridSpec(
        num_scalar_prefetch=1,
        grid=(L_TC // TC_BLOCK,),
        in_specs=[pl.BlockSpec((NUM_EMB, D_MODEL), lambda i, *_: (0, 0))],
        out_specs=pl.BlockSpec((TC_BLOCK, 1, D_MODEL), lambda i, *_: (i, 0, 0)),
    ),
    out_shape=jax.ShapeDtypeStruct((L_TC, 1, D_MODEL), jnp.float32),
)


def kernel(x, num_frame, embedding):
    L, N, D = x.shape
    ntxt = jnp.asarray(L - num_frame, dtype=jnp.int32)
    ntxt_vec = jnp.full((LANES,), ntxt, dtype=jnp.int32)
    out_sc = _emb_lookup_sc(embedding, ntxt_vec)
    out_tc = _emb_lookup_tc(ntxt.reshape(1), embedding)
    return jnp.concatenate([out_sc, out_tc], axis=0)


# speedup vs baseline: 3.3569x; 3.3569x over previous
import functools

import jax
import jax.numpy as jnp
from jax import lax
from jax.experimental import pallas as pl
from jax.experimental.pallas import tpu as pltpu
from jax.experimental.pallas import tpu_sc as plsc

D_MODEL = 1024
L_SEQ = 4096
NUM_EMB = 5
TEXT_ID = 0
VISUAL_ID = 3

NUM_CORES = 2
NUM_SUBCORES = 16
LANES = 16
NUM_WORKERS = NUM_CORES * NUM_SUBCORES
ROWS_PER_WORKER = L_SEQ // NUM_WORKERS
CHUNK = 16
NCHUNK = ROWS_PER_WORKER // CHUNK
SLICES = D_MODEL // LANES

_MESH = plsc.VectorSubcoreMesh(core_axis_name="c", subcore_axis_name="s")


@functools.partial(
    pl.kernel,
    out_type=jax.ShapeDtypeStruct((L_SEQ, 1, D_MODEL), jnp.float32),
    mesh=_MESH,
    scratch_types=[
        pltpu.VMEM((NUM_EMB, D_MODEL), jnp.float32),
        pltpu.VMEM((LANES,), jnp.int32),
        pltpu.VMEM((CHUNK, 1, D_MODEL), jnp.float32),
        pltpu.VMEM((CHUNK, 1, D_MODEL), jnp.float32),
        pltpu.SemaphoreType.DMA,
        pltpu.SemaphoreType.DMA,
        pltpu.SemaphoreType.DMA,
    ],
)
def _emb_lookup(table_hbm, ntxt_hbm, out_hbm, tab_v, ntxt_v, buf0, buf1,
                sem0, sem1, sem_in):
    wid = lax.axis_index("s") * NUM_CORES + lax.axis_index("c")
    base = wid * ROWS_PER_WORKER
    tab_dma = pltpu.async_copy(table_hbm, tab_v, sem_in)
    pltpu.sync_copy(ntxt_hbm, ntxt_v)
    ntxt = ntxt_v[...]
    tab_dma.wait()

    def fill(buf, c):
        cbase = base + c * CHUNK
        conds = [jnp.full((LANES,), cbase + r, jnp.int32) < ntxt
                 for r in range(CHUNK)]

        def body(s, carry):
            off = s * LANES
            e0 = tab_v[TEXT_ID, pl.ds(off, LANES)]
            e3 = tab_v[VISUAL_ID, pl.ds(off, LANES)]
            for r in range(CHUNK):
                buf[r, 0, pl.ds(off, LANES)] = jnp.where(conds[r], e0, e3)
            return carry
        lax.fori_loop(0, SLICES, body, 0)

    bufs = (buf0, buf1)
    sems = (sem0, sem1)
    handles = [None, None]
    for c in range(NCHUNK):
        b = c % 2
        if handles[b] is not None:
            handles[b].wait()
        fill(bufs[b], c)
        handles[b] = pltpu.async_copy(
            bufs[b], out_hbm.at[pl.ds(base + c * CHUNK, CHUNK)], sems[b])
    for h in handles:
        if h is not None:
            h.wait()


def kernel(x, num_frame, embedding):
    L, N, D = x.shape
    num_txt = jnp.full((LANES,), L - num_frame, dtype=jnp.int32)
    return _emb_lookup(embedding, num_txt)
